# Initial kernel scaffold; baseline (speedup 1.0000x reference)
import jax
import jax.numpy as jnp
from jax.experimental import pallas as pl


def kernel(adj_matrix, emb1, W1, b1, W2, b2):
    raise NotImplementedError("write your pallas kernel here")


# speedup vs baseline: 1.1529x; 1.1529x over previous
import jax
import jax.numpy as jnp
from jax.experimental import pallas as pl
from jax.experimental.pallas import tpu as pltpu

N, FEAT, HID, OUT = 4096, 256, 256, 128
BM = 512
M_BLOCKS = N // BM


def _body(adj_ref, emb1_ref, w1_ref, b1_ref, w2_ref, b2_ref, out_ref,
          xw1_scr, xw2_scr):
    i = pl.program_id(0)
    m = jax.lax.rem(i, M_BLOCKS)

    @pl.when(i == 0)
    def _():
        e = emb1_ref[...].astype(jnp.bfloat16)
        w = w1_ref[...].astype(jnp.bfloat16)
        xw1_scr[...] = jnp.dot(
            e, w, preferred_element_type=jnp.float32).astype(jnp.bfloat16)

    a = adj_ref[...].astype(jnp.bfloat16)

    @pl.when(i < M_BLOCKS)
    def _():
        acc = jnp.dot(a, xw1_scr[...], preferred_element_type=jnp.float32)
        x1 = jnp.maximum(acc + b1_ref[...], 0.0).astype(jnp.bfloat16)
        w2 = w2_ref[...].astype(jnp.bfloat16)
        xw2_scr[pl.ds(m * BM, BM), :] = jnp.dot(
            x1, w2, preferred_element_type=jnp.float32).astype(jnp.bfloat16)
        out_ref[...] = jnp.zeros_like(out_ref)

    @pl.when(i >= M_BLOCKS)
    def _():
        out_ref[...] = jnp.dot(
            a, xw2_scr[...], preferred_element_type=jnp.float32) + b2_ref[...]


def kernel(adj_matrix, emb1, W1, b1, W2, b2):
    b1r = b1.reshape(1, HID)
    b2r = b2.reshape(1, OUT)
    return pl.pallas_call(
        _body,
        grid=(2 * M_BLOCKS,),
        in_specs=[
            pl.BlockSpec((BM, N), lambda i: (jax.lax.rem(i, M_BLOCKS), 0)),
            pl.BlockSpec((N, FEAT), lambda i: (0, 0)),
            pl.BlockSpec((FEAT, HID), lambda i: (0, 0)),
            pl.BlockSpec((1, HID), lambda i: (0, 0)),
            pl.BlockSpec((HID, OUT), lambda i: (0, 0)),
            pl.BlockSpec((1, OUT), lambda i: (0, 0)),
        ],
        out_specs=pl.BlockSpec((BM, OUT), lambda i: (jax.lax.rem(i, M_BLOCKS), 0)),
        out_shape=jax.ShapeDtypeStruct((N, OUT), jnp.float32),
        scratch_shapes=[
            pltpu.VMEM((N, FEAT), jnp.bfloat16),
            pltpu.VMEM((N, OUT), jnp.bfloat16),
        ],
    )(adj_matrix, emb1, W1, b1r, W2, b2r)


# speedup vs baseline: 1.3237x; 1.1482x over previous
import jax
import jax.numpy as jnp
from jax.experimental import pallas as pl
from jax.experimental.pallas import tpu as pltpu

N, FEAT, HID, OUT = 4096, 256, 256, 128
BM = 256
M_BLOCKS = N // BM


def _body(adj_ref, emb1_ref, w1_ref, b1_ref, w2_ref, b2_ref, out_ref,
          adj_scr, xw1_scr, xw2_scr):
    i = pl.program_id(0)
    m = jax.lax.rem(i, M_BLOCKS)

    @pl.when(i == 0)
    def _():
        e = emb1_ref[...].astype(jnp.bfloat16)
        w = w1_ref[...].astype(jnp.bfloat16)
        xw1_scr[...] = jnp.dot(
            e, w, preferred_element_type=jnp.float32).astype(jnp.bfloat16)

    @pl.when(i < M_BLOCKS)
    def _():
        a = adj_ref[...].astype(jnp.bfloat16)
        adj_scr[pl.ds(m * BM, BM), :] = a
        acc = jnp.dot(a, xw1_scr[...], preferred_element_type=jnp.float32)
        x1 = jnp.maximum(acc + b1_ref[...], 0.0).astype(jnp.bfloat16)
        w2 = w2_ref[...].astype(jnp.bfloat16)
        xw2_scr[pl.ds(m * BM, BM), :] = jnp.dot(
            x1, w2, preferred_element_type=jnp.float32).astype(jnp.bfloat16)
        out_ref[...] = jnp.zeros_like(out_ref)

    @pl.when(i >= M_BLOCKS)
    def _():
        a = adj_scr[pl.ds(m * BM, BM), :]
        out_ref[...] = jnp.dot(
            a, xw2_scr[...], preferred_element_type=jnp.float32) + b2_ref[...]


def kernel(adj_matrix, emb1, W1, b1, W2, b2):
    b1r = b1.reshape(1, HID)
    b2r = b2.reshape(1, OUT)
    return pl.pallas_call(
        _body,
        grid=(2 * M_BLOCKS,),
        in_specs=[
            pl.BlockSpec((BM, N), lambda i: (jnp.minimum(i, M_BLOCKS - 1), 0)),
            pl.BlockSpec((N, FEAT), lambda i: (0, 0)),
            pl.BlockSpec((FEAT, HID), lambda i: (0, 0)),
            pl.BlockSpec((1, HID), lambda i: (0, 0)),
            pl.BlockSpec((HID, OUT), lambda i: (0, 0)),
            pl.BlockSpec((1, OUT), lambda i: (0, 0)),
        ],
        out_specs=pl.BlockSpec((BM, OUT), lambda i: (jax.lax.rem(i, M_BLOCKS), 0)),
        out_shape=jax.ShapeDtypeStruct((N, OUT), jnp.float32),
        scratch_shapes=[
            pltpu.VMEM((N, N), jnp.bfloat16),
            pltpu.VMEM((N, FEAT), jnp.bfloat16),
            pltpu.VMEM((N, OUT), jnp.bfloat16),
        ],
    )(adj_matrix, emb1, W1, b1r, W2, b2r)


# speedup vs baseline: 1.5227x; 1.1503x over previous
import jax
import jax.numpy as jnp
from jax.experimental import pallas as pl
from jax.experimental.pallas import tpu as pltpu

N, FEAT, HID, OUT = 4096, 256, 256, 128
BM = 512
M_BLOCKS = N // BM


def _body(adj_ref, emb1_ref, w1_ref, b1_ref, w2_ref, b2_ref, out_ref,
          adj_scr, xw1_scr, xw2_scr):
    i = pl.program_id(0)
    m = jax.lax.rem(i, M_BLOCKS)

    @pl.when(i == 0)
    def _():
        e = emb1_ref[...].astype(jnp.bfloat16)
        w = w1_ref[...].astype(jnp.bfloat16)
        xw1_scr[...] = jnp.dot(
            e, w, preferred_element_type=jnp.float32).astype(jnp.bfloat16)

    @pl.when(i < M_BLOCKS)
    def _():
        a = adj_ref[...].astype(jnp.bfloat16)
        adj_scr[pl.ds(m * BM, BM), :] = a
        acc = jnp.dot(a, xw1_scr[...], preferred_element_type=jnp.float32)
        x1 = jnp.maximum(acc + b1_ref[...], 0.0).astype(jnp.bfloat16)
        w2 = w2_ref[...].astype(jnp.bfloat16)
        xw2_scr[pl.ds(m * BM, BM), :] = jnp.dot(
            x1, w2, preferred_element_type=jnp.float32).astype(jnp.bfloat16)

    @pl.when(i >= M_BLOCKS)
    def _():
        a = adj_scr[pl.ds(m * BM, BM), :]
        out_ref[...] = jnp.dot(
            a, xw2_scr[...], preferred_element_type=jnp.float32) + b2_ref[...]


def kernel(adj_matrix, emb1, W1, b1, W2, b2):
    b1r = b1.reshape(1, HID)
    b2r = b2.reshape(1, OUT)
    return pl.pallas_call(
        _body,
        grid=(2 * M_BLOCKS,),
        in_specs=[
            pl.BlockSpec((BM, N), lambda i: (jnp.minimum(i, M_BLOCKS - 1), 0)),
            pl.BlockSpec((N, FEAT), lambda i: (0, 0)),
            pl.BlockSpec((FEAT, HID), lambda i: (0, 0)),
            pl.BlockSpec((1, HID), lambda i: (0, 0)),
            pl.BlockSpec((HID, OUT), lambda i: (0, 0)),
            pl.BlockSpec((1, OUT), lambda i: (0, 0)),
        ],
        out_specs=pl.BlockSpec((BM, OUT), lambda i: (jax.lax.rem(i, M_BLOCKS), 0)),
        out_shape=jax.ShapeDtypeStruct((N, OUT), jnp.float32),
        scratch_shapes=[
            pltpu.VMEM((N, N), jnp.bfloat16),
            pltpu.VMEM((N, FEAT), jnp.bfloat16),
            pltpu.VMEM((N, OUT), jnp.bfloat16),
        ],
    )(adj_matrix, emb1, W1, b1r, W2, b2r)
